# baseline (speedup 1.0000x reference)
import functools

import jax
import jax.numpy as jnp
from jax import lax
from jax.experimental import pallas as pl
from jax.experimental.pallas import tpu as pltpu
from jax.experimental.pallas import tpu_sc as plsc

N = 10000
D = 128
H = 128
C = 40
E = 320000
EPS = 1e-5

NC = 2
NS = 16
NW = NC * NS
EP = E // NW
CH = 80
NCHUNK = EP // CH
RPT = 624
ZR = 48
TAIL = N - NS * RPT


def _mm_body(x_ref, w_ref, o_ref):
    o_ref[...] = jnp.dot(x_ref[...], w_ref[...],
                         preferred_element_type=jnp.float32)


def _tc_matmul(x, w):
    return pl.pallas_call(
        _mm_body,
        out_shape=jax.ShapeDtypeStruct((x.shape[0], w.shape[1]), jnp.float32),
    )(x, w)


def _bn_mm_body(sa_ref, sb_ref, b_ref, g_ref, be_ref, w_ref, o_ref):
    z = sa_ref[...] + sb_ref[...] + b_ref[...]
    mu = jnp.mean(z, axis=0, keepdims=True)
    zc = z - mu
    var = jnp.mean(zc * zc, axis=0, keepdims=True)
    zn = g_ref[...] * zc * lax.rsqrt(var + EPS) + be_ref[...]
    zn = jnp.maximum(zn, 0.0)
    o_ref[...] = jnp.dot(zn, w_ref[...], preferred_element_type=jnp.float32)


def _tc_bn_mm(sa, sb, b, g, be, w):
    return pl.pallas_call(
        _bn_mm_body,
        out_shape=jax.ShapeDtypeStruct((sa.shape[0], w.shape[1]), jnp.float32),
    )(sa, sb, b.reshape(1, -1), g.reshape(1, -1), be.reshape(1, -1), w)


def _final_body(sa_ref, sb_ref, b_ref, o_ref):
    z = sa_ref[...] + sb_ref[...] + b_ref[...]
    col = lax.broadcasted_iota(jnp.int32, z.shape, 1)
    z = jnp.where(col < C, z, -1e30)
    zmax = jnp.max(z, axis=1, keepdims=True)
    zs = z - zmax
    lse = jnp.log(jnp.sum(jnp.exp(zs), axis=1, keepdims=True))
    o_ref[...] = (zs - lse)[:, :C]


def _tc_final(sa, sb, b):
    return pl.pallas_call(
        _final_body,
        out_shape=jax.ShapeDtypeStruct((sa.shape[0], C), jnp.float32),
    )(sa, sb, b)


def _make_spmm(dw):
    grp = dw // 16
    mesh = plsc.VectorSubcoreMesh(core_axis_name="c", subcore_axis_name="s",
                                  num_cores=NC, num_subcores=NS)

    @functools.partial(
        pl.kernel,
        out_type=jax.ShapeDtypeStruct((NC, N, dw), jnp.float32),
        mesh=mesh,
        scratch_types=[
            pltpu.VMEM((3, 3, CH), jnp.int32),
            pltpu.VMEM((3, CH), jnp.int32),
            pltpu.VMEM((3, CH, dw), jnp.float32),
            pltpu.VMEM_SHARED((N, dw), jnp.float32),
            [pltpu.SemaphoreType.DMA] * 3,
            [pltpu.SemaphoreType.DMA] * 3,
            [pltpu.SemaphoreType.DMA] * 3,
        ],
    )
    def spmm(y_hbm, ei_hbm, out_hbm,
             eb, dstb, rows, acc, semE, semG, semS):
        cid = lax.axis_index("c")
        sid = lax.axis_index("s")
        wid = sid * NC + cid

        def estart(p, j):
            return pltpu.async_copy(ei_hbm.at[wid, j], eb.at[p], semE[p])

        def ewait(p, j):
            pltpu.make_async_copy(ei_hbm.at[wid, j], eb.at[p],
                                  semE[p]).wait()

        def gstart(p):
            return pltpu.async_copy(y_hbm.at[eb.at[p, 0]], rows.at[p],
                                    semG[p])

        def gwait(p):
            pltpu.make_async_copy(y_hbm.at[eb.at[p, 0]], rows.at[p],
                                  semG[p]).wait()

        def sstart(p):
            return pltpu.async_copy(rows.at[p], acc.at[dstb.at[p]],
                                    semS[p], add=True)

        def swait(p):
            pltpu.make_async_copy(rows.at[p], acc.at[dstb.at[p]],
                                  semS[p]).wait()

        def scale(p):
            def egroup(b, c2):
                wi16 = eb[p, 2, pl.ds(b * 16, 16)]
                w16 = lax.bitcast_convert_type(wi16, jnp.float32)
                dstb[p, pl.ds(b * 16, 16)] = eb[p, 1, pl.ds(b * 16, 16)]
                for l in range(16):
                    e = b * 16 + l
                    w = w16[l]
                    for g in range(grp):
                        sl = pl.ds(g * 16, 16)
                        rows[p, e, sl] = rows[p, e, sl] * w
                return c2

            lax.fori_loop(0, CH // 16, egroup, 0)

        def step(j, p, first, last_e, last_g):
            pn = (p + 2) % 3
            if not last_g:
                ewait(pn, j + 2)
            if not first:
                swait(pn)
            if not last_g:
                gstart(pn)
            gwait(p)
            scale(p)
            sstart(p)
            if not last_e:
                estart(p, j + 3)

        zero16 = jnp.zeros((16,), jnp.float32)

        def zrow(i, carry):
            for g in range(grp):
                rows[0, i, pl.ds(g * 16, 16)] = zero16
            return carry

        lax.fori_loop(0, CH, zrow, 0)
        base = sid * RPT
        for k in range(RPT // CH):
            pltpu.sync_copy(rows.at[0],
                            acc.at[pl.ds(base + k * CH, CH)])
        rem = RPT - (RPT // CH) * CH
        pltpu.sync_copy(rows.at[0, pl.ds(0, rem)],
                        acc.at[pl.ds(base + RPT - rem, rem)])

        @pl.when(sid == 0)
        def _():
            pltpu.sync_copy(rows.at[0, pl.ds(0, TAIL)],
                            acc.at[pl.ds(NS * RPT, TAIL)])

        plsc.subcore_barrier()

        estart(0, 0)
        estart(1, 1)
        estart(2, 2)
        ewait(0, 0)
        gstart(0)
        ewait(1, 1)
        gstart(1)

        step(0, 0, True, False, False)
        step(1, 1, False, False, False)
        step(2, 2, False, False, False)

        def pipe(k, carry):
            j0 = 3 * k + 3

            def parity(off):
                step(j0 + off, off, False,
                     last_e=False, last_g=False)

            parity(0)
            parity(1)
            p = 2
            j = j0 + 2
            pn = (p + 2) % 3
            ewait(pn, j + 2)
            swait(pn)
            gstart(pn)
            gwait(p)
            scale(p)
            sstart(p)

            @pl.when(j + 3 < NCHUNK)
            def _():
                estart(p, j + 3)

            return carry

        lax.fori_loop(0, (NCHUNK - 5) // 3, pipe, 0)

        step(NCHUNK - 2, 0, False, True, True)
        step(NCHUNK - 1, 1, False, True, True)
        swait(1)
        plsc.subcore_barrier()

        for k in range(RPT // ZR):
            sl = pl.ds(sid * RPT + k * ZR, ZR)
            pltpu.sync_copy(acc.at[sl], out_hbm.at[cid, sl])

        @pl.when(sid == 0)
        def _():
            sl = pl.ds(NS * RPT, TAIL)
            pltpu.sync_copy(acc.at[sl], out_hbm.at[cid, sl])

    return spmm


_spmm128 = _make_spmm(128)


def kernel(x, edge_index, edge_weight, W1, b1, W2, b2, W3, b3, g1, be1,
           g2, be2):
    dst = edge_index[0].astype(jnp.int32).reshape(NW, NCHUNK, CH)
    src = edge_index[1].astype(jnp.int32).reshape(NW, NCHUNK, CH)
    wbits = lax.bitcast_convert_type(
        edge_weight, jnp.int32).reshape(NW, NCHUNK, CH)
    ei = jnp.stack([src, dst, wbits], axis=2)

    y1 = _tc_matmul(x, W1)
    s1 = _spmm128(y1, ei)
    y2 = _tc_bn_mm(s1[0], s1[1], b1, g1, be1, W2)
    s2 = _spmm128(y2, ei)
    W3p = jnp.zeros((H, H), jnp.float32).at[:, :C].set(W3)
    y3 = _tc_bn_mm(s2[0], s2[1], b2, g2, be2, W3p)
    s3 = _spmm128(y3, ei)
    b3p = jnp.zeros((1, H), jnp.float32).at[0, :C].set(b3)
    return _tc_final(s3[0], s3[1], b3p)


# speedup vs baseline: 1.0206x; 1.0206x over previous
import functools

import jax
import jax.numpy as jnp
from jax import lax
from jax.experimental import pallas as pl
from jax.experimental.pallas import tpu as pltpu
from jax.experimental.pallas import tpu_sc as plsc

N = 10000
D = 128
H = 128
C = 40
E = 320000
EPS = 1e-5

NC = 2
NS = 16
NW = NC * NS
EP = E // NW
CH = 80
NCHUNK = EP // CH
RPT = 624
ZR = 48
TAIL = N - NS * RPT


def _mm_body(x_ref, w_ref, o_ref):
    o_ref[...] = jnp.dot(x_ref[...], w_ref[...],
                         preferred_element_type=jnp.float32)


def _tc_matmul(x, w):
    return pl.pallas_call(
        _mm_body,
        out_shape=jax.ShapeDtypeStruct((x.shape[0], w.shape[1]), jnp.float32),
    )(x, w)


def _bn_mm_body(sa_ref, sb_ref, b_ref, g_ref, be_ref, w_ref, o_ref):
    z = sa_ref[...] + sb_ref[...] + b_ref[...]
    mu = jnp.mean(z, axis=0, keepdims=True)
    zc = z - mu
    var = jnp.mean(zc * zc, axis=0, keepdims=True)
    zn = g_ref[...] * zc * lax.rsqrt(var + EPS) + be_ref[...]
    zn = jnp.maximum(zn, 0.0)
    o_ref[...] = jnp.dot(zn, w_ref[...], preferred_element_type=jnp.float32)


def _tc_bn_mm(sa, sb, b, g, be, w):
    return pl.pallas_call(
        _bn_mm_body,
        out_shape=jax.ShapeDtypeStruct((sa.shape[0], w.shape[1]), jnp.float32),
    )(sa, sb, b.reshape(1, -1), g.reshape(1, -1), be.reshape(1, -1), w)


def _final_body(sa_ref, sb_ref, b_ref, o_ref):
    z = sa_ref[...] + sb_ref[...] + b_ref[...]
    col = lax.broadcasted_iota(jnp.int32, z.shape, 1)
    z = jnp.where(col < C, z, -1e30)
    zmax = jnp.max(z, axis=1, keepdims=True)
    zs = z - zmax
    lse = jnp.log(jnp.sum(jnp.exp(zs), axis=1, keepdims=True))
    o_ref[...] = (zs - lse)[:, :C]


def _tc_final(sa, sb, b):
    return pl.pallas_call(
        _final_body,
        out_shape=jax.ShapeDtypeStruct((sa.shape[0], C), jnp.float32),
    )(sa, sb, b)


def _make_spmm(dw):
    grp = dw // 16
    mesh = plsc.VectorSubcoreMesh(core_axis_name="c", subcore_axis_name="s",
                                  num_cores=NC, num_subcores=NS)

    @functools.partial(
        pl.kernel,
        out_type=jax.ShapeDtypeStruct((NC, N, dw), jnp.float32),
        mesh=mesh,
        scratch_types=[
            pltpu.VMEM((3, 3, CH), jnp.int32),
            pltpu.VMEM((3, CH), jnp.int32),
            pltpu.VMEM((3, CH, dw), jnp.float32),
            pltpu.VMEM_SHARED((N, dw), jnp.float32),
            [pltpu.SemaphoreType.DMA] * 3,
            [pltpu.SemaphoreType.DMA] * 3,
            [pltpu.SemaphoreType.DMA] * 3,
        ],
    )
    def spmm(y_hbm, ei_hbm, out_hbm,
             eb, dstb, rows, acc, semE, semG, semS):
        cid = lax.axis_index("c")
        sid = lax.axis_index("s")
        wid = sid * NC + cid

        def estart(p, j):
            return pltpu.async_copy(ei_hbm.at[wid, j], eb.at[p], semE[p])

        def ewait(p, j):
            pltpu.make_async_copy(ei_hbm.at[wid, j], eb.at[p],
                                  semE[p]).wait()

        def gstart(p):
            return pltpu.async_copy(y_hbm.at[eb.at[p, 0]], rows.at[p],
                                    semG[p])

        def gwait(p):
            pltpu.make_async_copy(y_hbm.at[eb.at[p, 0]], rows.at[p],
                                  semG[p]).wait()

        def sstart(p):
            return pltpu.async_copy(rows.at[p], acc.at[dstb.at[p]],
                                    semS[p], add=True)

        def swait(p):
            pltpu.make_async_copy(rows.at[p], acc.at[dstb.at[p]],
                                  semS[p]).wait()

        def scale(p):
            def egroup(b, c2):
                wi16 = eb[p, 2, pl.ds(b * 16, 16)]
                w16 = lax.bitcast_convert_type(wi16, jnp.float32)
                dstb[p, pl.ds(b * 16, 16)] = eb[p, 1, pl.ds(b * 16, 16)]
                for l in range(16):
                    e = b * 16 + l
                    w = w16[l]
                    for g in range(grp):
                        sl = pl.ds(g * 16, 16)
                        rows[p, e, sl] = rows[p, e, sl] * w
                return c2

            lax.fori_loop(0, CH // 16, egroup, 0)

        def step(j, p, first, last_e, last_g):
            pn = (p + 2) % 3
            if not last_g:
                ewait(pn, j + 2)
            if not first:
                swait(pn)
            if not last_g:
                gstart(pn)
            gwait(p)
            scale(p)
            sstart(p)
            if not last_e:
                estart(p, j + 3)

        zero16 = jnp.zeros((16,), jnp.float32)

        def zrow(i, carry):
            for g in range(grp):
                rows[0, i, pl.ds(g * 16, 16)] = zero16
            return carry

        estart(0, 0)
        estart(1, 1)
        estart(2, 2)

        lax.fori_loop(0, CH, zrow, 0)
        base = sid * RPT

        _REM = RPT - (RPT // CH) * CH

        def zpair(k):
            if k < RPT // CH:
                return rows.at[0], acc.at[pl.ds(base + k * CH, CH)]
            return (rows.at[0, pl.ds(0, _REM)],
                    acc.at[pl.ds(base + RPT - _REM, _REM)])

        for k in range(RPT // CH + 1):
            src, dst = zpair(k)
            pltpu.async_copy(src, dst, semS[0])
        for k in range(RPT // CH + 1):
            src, dst = zpair(k)
            pltpu.make_async_copy(src, dst, semS[0]).wait()

        @pl.when(sid == 0)
        def _():
            pltpu.sync_copy(rows.at[0, pl.ds(0, TAIL)],
                            acc.at[pl.ds(NS * RPT, TAIL)])

        plsc.subcore_barrier()

        ewait(0, 0)
        gstart(0)
        ewait(1, 1)
        gstart(1)

        step(0, 0, True, False, False)
        step(1, 1, False, False, False)
        step(2, 2, False, False, False)

        def pipe(k, carry):
            j0 = 3 * k + 3

            def parity(off):
                step(j0 + off, off, False,
                     last_e=False, last_g=False)

            parity(0)
            parity(1)
            p = 2
            j = j0 + 2
            pn = (p + 2) % 3
            ewait(pn, j + 2)
            swait(pn)
            gstart(pn)
            gwait(p)
            scale(p)
            sstart(p)

            @pl.when(j + 3 < NCHUNK)
            def _():
                estart(p, j + 3)

            return carry

        lax.fori_loop(0, (NCHUNK - 5) // 3, pipe, 0)

        step(NCHUNK - 2, 0, False, True, True)
        step(NCHUNK - 1, 1, False, True, True)
        swait(1)
        plsc.subcore_barrier()

        for k in range(RPT // ZR):
            sl = pl.ds(sid * RPT + k * ZR, ZR)
            pltpu.async_copy(acc.at[sl], out_hbm.at[cid, sl], semS[1])

        @pl.when(sid == 0)
        def _():
            sl = pl.ds(NS * RPT, TAIL)
            pltpu.sync_copy(acc.at[sl], out_hbm.at[cid, sl])

        for k in range(RPT // ZR):
            sl = pl.ds(sid * RPT + k * ZR, ZR)
            pltpu.make_async_copy(acc.at[sl], out_hbm.at[cid, sl],
                                  semS[1]).wait()

    return spmm


_spmm128 = _make_spmm(128)


def kernel(x, edge_index, edge_weight, W1, b1, W2, b2, W3, b3, g1, be1,
           g2, be2):
    dst = edge_index[0].astype(jnp.int32).reshape(NW, NCHUNK, CH)
    src = edge_index[1].astype(jnp.int32).reshape(NW, NCHUNK, CH)
    wbits = lax.bitcast_convert_type(
        edge_weight, jnp.int32).reshape(NW, NCHUNK, CH)
    ei = jnp.stack([src, dst, wbits], axis=2)

    y1 = _tc_matmul(x, W1)
    s1 = _spmm128(y1, ei)
    y2 = _tc_bn_mm(s1[0], s1[1], b1, g1, be1, W2)
    s2 = _spmm128(y2, ei)
    W3p = jnp.zeros((H, H), jnp.float32).at[:, :C].set(W3)
    y3 = _tc_bn_mm(s2[0], s2[1], b2, g2, be2, W3p)
    s3 = _spmm128(y3, ei)
    b3p = jnp.zeros((1, H), jnp.float32).at[0, :C].set(b3)
    return _tc_final(s3[0], s3[1], b3p)


# speedup vs baseline: 1.0300x; 1.0091x over previous
import functools

import jax
import jax.numpy as jnp
from jax import lax
from jax.experimental import pallas as pl
from jax.experimental.pallas import tpu as pltpu
from jax.experimental.pallas import tpu_sc as plsc

N = 10000
D = 128
H = 128
C = 40
E = 320000
EPS = 1e-5

NC = 2
NS = 16
NW = NC * NS
EP = E // NW
CH = 80
NCHUNK = EP // CH
RPT = 624
ZR = 48
TAIL = N - NS * RPT


def _mm_body(x_ref, w_ref, o_ref):
    o_ref[...] = jnp.dot(x_ref[...], w_ref[...],
                         preferred_element_type=jnp.float32)


def _tc_matmul(x, w):
    return pl.pallas_call(
        _mm_body,
        out_shape=jax.ShapeDtypeStruct((x.shape[0], w.shape[1]), jnp.float32),
    )(x, w)


def _bn_mm_body(sa_ref, sb_ref, b_ref, g_ref, be_ref, w_ref, o_ref):
    z = sa_ref[...] + sb_ref[...] + b_ref[...]
    mu = jnp.mean(z, axis=0, keepdims=True)
    zc = z - mu
    var = jnp.mean(zc * zc, axis=0, keepdims=True)
    zn = g_ref[...] * zc * lax.rsqrt(var + EPS) + be_ref[...]
    zn = jnp.maximum(zn, 0.0)
    o_ref[...] = jnp.dot(zn, w_ref[...], preferred_element_type=jnp.float32)


def _tc_bn_mm(sa, sb, b, g, be, w):
    return pl.pallas_call(
        _bn_mm_body,
        out_shape=jax.ShapeDtypeStruct((sa.shape[0], w.shape[1]), jnp.float32),
    )(sa, sb, b.reshape(1, -1), g.reshape(1, -1), be.reshape(1, -1), w)


def _final_body(sa_ref, sb_ref, b_ref, o_ref):
    z = sa_ref[...] + sb_ref[...] + b_ref[...]
    col = lax.broadcasted_iota(jnp.int32, z.shape, 1)
    z = jnp.where(col < C, z, -1e30)
    zmax = jnp.max(z, axis=1, keepdims=True)
    zs = z - zmax
    lse = jnp.log(jnp.sum(jnp.exp(zs), axis=1, keepdims=True))
    o_ref[...] = (zs - lse)[:, :C]


def _tc_final(sa, sb, b):
    return pl.pallas_call(
        _final_body,
        out_shape=jax.ShapeDtypeStruct((sa.shape[0], C), jnp.float32),
    )(sa, sb, b)


def _make_spmm(dw):
    grp = dw // 16
    mesh = plsc.VectorSubcoreMesh(core_axis_name="c", subcore_axis_name="s",
                                  num_cores=NC, num_subcores=NS)

    @functools.partial(
        pl.kernel,
        out_type=jax.ShapeDtypeStruct((NC, N, dw), jnp.float32),
        mesh=mesh,
        scratch_types=[
            pltpu.VMEM((3, 3, CH), jnp.int32),
            pltpu.VMEM((3, CH), jnp.int32),
            pltpu.VMEM((3, CH, dw), jnp.float32),
            pltpu.VMEM_SHARED((N, dw), jnp.float32),
            [pltpu.SemaphoreType.DMA] * 3,
            [pltpu.SemaphoreType.DMA] * 3,
            [pltpu.SemaphoreType.DMA] * 3,
        ],
    )
    def spmm(y_hbm, ei_hbm, out_hbm,
             eb, dstb, rows, acc, semE, semG, semS):
        cid = lax.axis_index("c")
        sid = lax.axis_index("s")
        wid = sid * NC + cid

        def estart(p, j):
            return pltpu.async_copy(ei_hbm.at[wid, j], eb.at[p], semE[p])

        def ewait(p, j):
            pltpu.make_async_copy(ei_hbm.at[wid, j], eb.at[p],
                                  semE[p]).wait()

        def gstart(p):
            return pltpu.async_copy(y_hbm.at[eb.at[p, 0]], rows.at[p],
                                    semG[p])

        def gwait(p):
            pltpu.make_async_copy(y_hbm.at[eb.at[p, 0]], rows.at[p],
                                  semG[p]).wait()

        def sstart(p):
            return pltpu.async_copy(rows.at[p], acc.at[dstb.at[p]],
                                    semS[p], add=True)

        def swait(p):
            pltpu.make_async_copy(rows.at[p], acc.at[dstb.at[p]],
                                  semS[p]).wait()

        def scale(p):
            def egroup(b, c2):
                wi16 = eb[p, 2, pl.ds(b * 16, 16)]
                w16 = lax.bitcast_convert_type(wi16, jnp.float32)
                dstb[p, pl.ds(b * 16, 16)] = eb[p, 1, pl.ds(b * 16, 16)]
                for l in range(16):
                    e = b * 16 + l
                    w = w16[l]
                    for g in range(grp):
                        sl = pl.ds(g * 16, 16)
                        rows[p, e, sl] = rows[p, e, sl] * w
                return c2

            lax.fori_loop(0, CH // 16, egroup, 0)

        def step(j, p, first, last_e, last_g):
            pn = (p + 2) % 3
            if not last_g:
                ewait(pn, j + 2)
            if not first:
                swait(pn)
            if not last_g:
                gstart(pn)
            gwait(p)
            scale(p)
            sstart(p)
            if not last_e:
                estart(p, j + 3)

        zero16 = jnp.zeros((16,), jnp.float32)

        def zrow(i, carry):
            for g in range(grp):
                rows[2, i, pl.ds(g * 16, 16)] = zero16
            return carry

        estart(0, 0)
        estart(1, 1)
        estart(2, 2)

        lax.fori_loop(0, CH, zrow, 0)
        base = sid * RPT

        _REM = RPT - (RPT // CH) * CH

        def zpair(k):
            if k < RPT // CH:
                return rows.at[2], acc.at[pl.ds(base + k * CH, CH)]
            return (rows.at[2, pl.ds(0, _REM)],
                    acc.at[pl.ds(base + RPT - _REM, _REM)])

        for k in range(RPT // CH + 1):
            src, dst = zpair(k)
            pltpu.async_copy(src, dst, semS[0])

        ewait(0, 0)
        gstart(0)
        ewait(1, 1)
        gstart(1)

        for k in range(RPT // CH + 1):
            src, dst = zpair(k)
            pltpu.make_async_copy(src, dst, semS[0]).wait()

        @pl.when(sid == 0)
        def _():
            pltpu.sync_copy(rows.at[2, pl.ds(0, TAIL)],
                            acc.at[pl.ds(NS * RPT, TAIL)])

        plsc.subcore_barrier()


        step(0, 0, True, False, False)
        step(1, 1, False, False, False)
        step(2, 2, False, False, False)

        def pipe(k, carry):
            j0 = 3 * k + 3

            def parity(off):
                step(j0 + off, off, False,
                     last_e=False, last_g=False)

            parity(0)
            parity(1)
            p = 2
            j = j0 + 2
            pn = (p + 2) % 3
            ewait(pn, j + 2)
            swait(pn)
            gstart(pn)
            gwait(p)
            scale(p)
            sstart(p)

            @pl.when(j + 3 < NCHUNK)
            def _():
                estart(p, j + 3)

            return carry

        lax.fori_loop(0, (NCHUNK - 5) // 3, pipe, 0)

        step(NCHUNK - 2, 0, False, True, True)
        step(NCHUNK - 1, 1, False, True, True)
        swait(1)
        plsc.subcore_barrier()

        for k in range(RPT // ZR):
            sl = pl.ds(sid * RPT + k * ZR, ZR)
            pltpu.async_copy(acc.at[sl], out_hbm.at[cid, sl], semS[1])

        @pl.when(sid == 0)
        def _():
            sl = pl.ds(NS * RPT, TAIL)
            pltpu.sync_copy(acc.at[sl], out_hbm.at[cid, sl])

        for k in range(RPT // ZR):
            sl = pl.ds(sid * RPT + k * ZR, ZR)
            pltpu.make_async_copy(acc.at[sl], out_hbm.at[cid, sl],
                                  semS[1]).wait()

    return spmm


_spmm128 = _make_spmm(128)


def kernel(x, edge_index, edge_weight, W1, b1, W2, b2, W3, b3, g1, be1,
           g2, be2):
    dst = edge_index[0].astype(jnp.int32).reshape(NW, NCHUNK, CH)
    src = edge_index[1].astype(jnp.int32).reshape(NW, NCHUNK, CH)
    wbits = lax.bitcast_convert_type(
        edge_weight, jnp.int32).reshape(NW, NCHUNK, CH)
    ei = jnp.stack([src, dst, wbits], axis=2)

    y1 = _tc_matmul(x, W1)
    s1 = _spmm128(y1, ei)
    y2 = _tc_bn_mm(s1[0], s1[1], b1, g1, be1, W2)
    s2 = _spmm128(y2, ei)
    W3p = jnp.zeros((H, H), jnp.float32).at[:, :C].set(W3)
    y3 = _tc_bn_mm(s2[0], s2[1], b2, g2, be2, W3p)
    s3 = _spmm128(y3, ei)
    b3p = jnp.zeros((1, H), jnp.float32).at[0, :C].set(b3)
    return _tc_final(s3[0], s3[1], b3p)
